# Initial kernel scaffold; baseline (speedup 1.0000x reference)
import jax
import jax.numpy as jnp
from jax.experimental import pallas as pl


def kernel(x, edge_index, s1_t1_w, s1_t1_b, s1_ch_W, s1_ch_b, s1_t2_w, s1_t2_b, s1_bn_g, s1_bn_b, s2_t1_w, s2_t1_b, s2_ch_W, s2_ch_b, s2_t2_w, s2_t2_b, s2_bn_g, s2_bn_b, fconv_w, fconv_b, fc_w, fc_b):
    raise NotImplementedError("write your pallas kernel here")


# speedup vs baseline: 1.0444x; 1.0444x over previous
import functools

import jax
import jax.numpy as jnp
from jax import lax
from jax.experimental import pallas as pl
from jax.experimental.pallas import tpu as pltpu

N_NODES = 10000
KSZ = 3


def _fc_body(h_ref, w_ref, o_ref):
    o_ref[...] = lax.dot_general(
        w_ref[...], h_ref[...], (((1,), (1,)), ((), ())),
        preferred_element_type=jnp.float32)


def _fc_matmul(h2, fc_w, fc_b):
    M = 8
    hp = jnp.zeros((M, N_NODES), jnp.float32).at[:2].set(h2)
    OB = 400
    out_t = pl.pallas_call(
        _fc_body,
        grid=(N_NODES // OB,),
        in_specs=[
            pl.BlockSpec((M, N_NODES), lambda i: (0, 0)),
            pl.BlockSpec((OB, N_NODES), lambda i: (i, 0)),
        ],
        out_specs=pl.BlockSpec((OB, M), lambda i: (i, 0)),
        out_shape=jax.ShapeDtypeStruct((N_NODES, M), jnp.float32),
    )(hp, fc_w)
    return out_t.T[:2] + fc_b[None, :]


def _tconv(X, w, b):
    B, T, N, C = X.shape
    H = w.shape[1]
    To = T - KSZ + 1
    Xs = jnp.stack([X[:, k:k + To] for k in range(KSZ)], axis=-1)
    Xs = Xs.reshape(B, To, N, C * KSZ)
    wm = jnp.transpose(w[:, :, :, 0, :], (0, 2, 3, 1)).reshape(3, C * KSZ, H)
    P = Xs @ wm[0] + b[0]
    Q = Xs @ wm[1] + b[1]
    R = Xs @ wm[2] + b[2]
    return jax.nn.relu(P * jax.nn.sigmoid(Q) + R)


def _cheb_norm(edge_index, num_nodes):
    row, col = edge_index[0], edge_index[1]
    m = row != col
    w = jnp.where(m, jnp.float32(1.0), jnp.float32(0.0))
    deg = jnp.zeros((num_nodes,), jnp.float32).at[row].add(w)
    dis = jnp.where(deg > 0, 1.0 / jnp.sqrt(deg), 0.0)
    nw = -dis[row] * w * dis[col]
    return row, col, nw


def _cheb(x, row, col, nw, W, b):
    def mv(z):
        msg = nw[None, None, :, None] * z[:, :, row, :]
        return jnp.zeros_like(z).at[:, :, col, :].add(msg)
    Tx0 = x
    out = Tx0 @ W[0]
    Tx1 = mv(Tx0)
    out = out + Tx1 @ W[1]
    Tx2 = 2.0 * mv(Tx1) - Tx0
    out = out + Tx2 @ W[2]
    return out + b


def _bn(x, g, bt, eps=1e-5):
    mean = x.mean(axis=(0, 1, 3), keepdims=True)
    var = ((x - mean) ** 2).mean(axis=(0, 1, 3), keepdims=True)
    y = (x - mean) / jnp.sqrt(var + eps)
    return y * g[None, None, :, None] + bt[None, None, :, None]


def _stconv(X, row, col, nw, t1w, t1b, chW, chb, t2w, t2b, bng, bnb):
    T0 = _tconv(X, t1w, t1b)
    T = jax.nn.relu(_cheb(T0, row, col, nw, chW, chb))
    T = _tconv(T, t2w, t2b)
    return _bn(T, bng, bnb)


def kernel(x, edge_index, s1_t1_w, s1_t1_b, s1_ch_W, s1_ch_b, s1_t2_w,
           s1_t2_b, s1_bn_g, s1_bn_b, s2_t1_w, s2_t1_b, s2_ch_W, s2_ch_b,
           s2_t2_w, s2_t2_b, s2_bn_g, s2_bn_b, fconv_w, fconv_b, fc_w, fc_b):
    row, col, nw = _cheb_norm(edge_index, N_NODES)
    h = _stconv(x, row, col, nw, s1_t1_w, s1_t1_b, s1_ch_W, s1_ch_b,
                s1_t2_w, s1_t2_b, s1_bn_g, s1_bn_b)
    h = jax.nn.relu(h)
    h = _stconv(h, row, col, nw, s2_t1_w, s2_t1_b, s2_ch_W, s2_ch_b,
                s2_t2_w, s2_t2_b, s2_bn_g, s2_bn_b)
    h = jax.nn.relu(h)
    h = h * fconv_w[0, 0, 0, 0] + fconv_b[0]
    h2 = h.reshape(h.shape[0], -1)
    return _fc_matmul(h2, fc_w, fc_b)


# speedup vs baseline: 28.5507x; 27.3377x over previous
import functools

import jax
import jax.numpy as jnp
from jax import lax
from jax.experimental import pallas as pl
from jax.experimental.pallas import tpu as pltpu
from jax.experimental.pallas import tpu_sc as plsc

N_NODES = 10000
NPAD = 10112
TRASH = 10000
KSZ = 3
KE = 128
NSUB = 16

@functools.cache
def _mesh():
    return plsc.VectorSubcoreMesh(core_axis_name="c", subcore_axis_name="s",
                                  num_cores=2, num_subcores=NSUB)


def _epad(E):
    g = 32 * KE
    return -(-E // g) * g


def _sc_mv_call(z_flat, row_flat, col_pad, zeros_pad, Fc, nchunks):
    per_core = nchunks // 2
    EP = row_flat.shape[0] // nchunks
    esub = EP // NSUB
    iters = esub // KE
    rps = NPAD // NSUB

    def body(z_ref, row_ref, col_ref, zero_ref, y_ref, acc, row_v, col_v,
             gbuf, sem):
        c = lax.axis_index("c")
        s = lax.axis_index("s")
        for jj in range(per_core):
            chunk = c * per_core + jj
            pltpu.sync_copy(zero_ref.at[pl.ds(s * rps, rps)],
                            acc.at[pl.ds(s * rps, rps)])
            plsc.subcore_barrier()

            def it(i, carry):
                eb = s * esub + i * KE
                pltpu.sync_copy(row_ref.at[pl.ds(chunk * EP + eb, KE)], row_v)
                pltpu.sync_copy(col_ref.at[pl.ds(eb, KE)], col_v)
                pltpu.async_copy(z_ref.at[row_v], gbuf, sem).wait()
                pltpu.sync_copy(gbuf, acc.at[col_v], add=True)
                return carry

            lax.fori_loop(0, iters, it, 0)
            plsc.subcore_barrier()
            pltpu.sync_copy(acc.at[pl.ds(s * rps, rps)],
                            y_ref.at[pl.ds(chunk * NPAD + s * rps, rps)])
            plsc.subcore_barrier()

    k = pl.kernel(
        body,
        out_type=jax.ShapeDtypeStruct((nchunks * NPAD, Fc), jnp.float32),
        mesh=_mesh(),
        scratch_types=[
            pltpu.VMEM_SHARED((NPAD, Fc), jnp.float32),
            pltpu.VMEM((KE,), jnp.int32),
            pltpu.VMEM((KE,), jnp.int32),
            pltpu.VMEM((KE, Fc), jnp.float32),
            pltpu.SemaphoreType.DMA,
        ],
    )
    return k(z_flat, row_flat, col_pad, zeros_pad)


def _sc_deg_call(row_deg, ones_in, zeros_deg):
    EP = row_deg.shape[0]
    esub = EP // (2 * NSUB)
    iters = esub // KE
    rps = NPAD // NSUB

    def body(rows_ref, ones_ref, zero_ref, y_ref, acc, row_v, vbuf, sem):
        c = lax.axis_index("c")
        s = lax.axis_index("s")
        wid = c * NSUB + s
        pltpu.sync_copy(zero_ref.at[pl.ds(s * rps, rps)],
                        acc.at[pl.ds(s * rps, rps)])
        pltpu.sync_copy(ones_ref, vbuf)
        plsc.subcore_barrier()

        def it(i, carry):
            eb = wid * esub + i * KE
            pltpu.sync_copy(rows_ref.at[pl.ds(eb, KE)], row_v)
            pltpu.sync_copy(vbuf, acc.at[row_v], add=True)
            return carry

        lax.fori_loop(0, iters, it, 0)
        plsc.subcore_barrier()
        pltpu.sync_copy(acc.at[pl.ds(s * rps, rps)],
                        y_ref.at[pl.ds(c * NPAD + s * rps, rps)])

    k = pl.kernel(
        body,
        out_type=jax.ShapeDtypeStruct((2 * NPAD, 128), jnp.float32),
        mesh=_mesh(),
        scratch_types=[
            pltpu.VMEM_SHARED((NPAD, 128), jnp.float32),
            pltpu.VMEM((KE,), jnp.int32),
            pltpu.VMEM((KE, 128), jnp.float32),
            pltpu.SemaphoreType.DMA,
        ],
    )
    return k(row_deg, ones_in, zeros_deg)


_NBLK = 2000


def _rowspec(cols):
    return pl.BlockSpec((_NBLK, cols), lambda i: (i, 0))


def _fullspec(r, c):
    return pl.BlockSpec((r, c), lambda i: (0, 0))


def _tconv_body(x_ref, gp, gq, gr, bp, bq, br, dis_ref, z_ref, zs_ref):
    X = x_ref[...]
    P = jnp.dot(X, gp[...], preferred_element_type=jnp.float32) + bp[...]
    Q = jnp.dot(X, gq[...], preferred_element_type=jnp.float32) + bq[...]
    R = jnp.dot(X, gr[...], preferred_element_type=jnp.float32) + br[...]
    Z = jnp.maximum(P * jax.nn.sigmoid(Q) + R, 0.0)
    z_ref[...] = Z
    zs_ref[...] = Z * dis_ref[...]


def _tconv_call(X, gp, gq, gr, bp, bq, br, dis):
    Ci = X.shape[1]
    F = gp.shape[1]
    return pl.pallas_call(
        _tconv_body,
        grid=(N_NODES // _NBLK,),
        in_specs=[_rowspec(Ci), _fullspec(Ci, F), _fullspec(Ci, F),
                  _fullspec(Ci, F), _fullspec(1, F), _fullspec(1, F),
                  _fullspec(1, F), _rowspec(1)],
        out_specs=[_rowspec(F), _rowspec(F)],
        out_shape=[jax.ShapeDtypeStruct((N_NODES, F), jnp.float32)] * 2,
    )(X, gp, gq, gr, bp, bq, br, dis)


def _scale_body(y_ref, s_ref, o_ref):
    o_ref[...] = y_ref[...] * s_ref[...]


def _scale_call(y, s):
    F = y.shape[1]
    return pl.pallas_call(
        _scale_body,
        grid=(N_NODES // _NBLK,),
        in_specs=[_rowspec(F), _rowspec(1)],
        out_specs=_rowspec(F),
        out_shape=jax.ShapeDtypeStruct((N_NODES, F), jnp.float32),
    )(y, s)


def _comb_body(z0_ref, y1_ref, y2_ref, dis_ref, wa, w1, w2, bch,
               g2p, g2q, g2r, b2p, b2q, b2r, g_ref, b_ref, o_ref):
    z0 = z0_ref[...]
    dis = dis_ref[...]
    tx1 = -dis * y1_ref[...]
    tx2 = (-2.0 * dis) * y2_ref[...] - z0
    out = jnp.dot(z0, wa[...], preferred_element_type=jnp.float32)
    out = out + jnp.dot(tx1, w1[...], preferred_element_type=jnp.float32)
    out = out + jnp.dot(tx2, w2[...], preferred_element_type=jnp.float32)
    C = jnp.maximum(out + bch[...], 0.0)
    P = jnp.dot(C, g2p[...], preferred_element_type=jnp.float32) + b2p[...]
    Q = jnp.dot(C, g2q[...], preferred_element_type=jnp.float32) + b2q[...]
    R = jnp.dot(C, g2r[...], preferred_element_type=jnp.float32) + b2r[...]
    D = jnp.maximum(P * jax.nn.sigmoid(Q) + R, 0.0)
    m = jnp.mean(D, axis=1, keepdims=True)
    v = jnp.mean((D - m) ** 2, axis=1, keepdims=True)
    Yn = (D - m) / jnp.sqrt(v + 1e-5) * g_ref[...] + b_ref[...]
    o_ref[...] = jnp.maximum(Yn, 0.0)


def _comb_call(z0, y1, y2, dis, wa, w1, w2, bch, g2p, g2q, g2r,
               b2p, b2q, b2r, g, b):
    F = z0.shape[1]
    Do = g2p.shape[1]
    return pl.pallas_call(
        _comb_body,
        grid=(N_NODES // _NBLK,),
        in_specs=[_rowspec(F), _rowspec(F), _rowspec(F), _rowspec(1),
                  _fullspec(F, F), _fullspec(F, F), _fullspec(F, F),
                  _fullspec(1, F), _fullspec(F, Do), _fullspec(F, Do),
                  _fullspec(F, Do), _fullspec(1, Do), _fullspec(1, Do),
                  _fullspec(1, Do), _rowspec(1), _rowspec(1)],
        out_specs=_rowspec(Do),
        out_shape=jax.ShapeDtypeStruct((N_NODES, Do), jnp.float32),
    )(z0, y1, y2, dis, wa, w1, w2, bch, g2p, g2q, g2r, b2p, b2q, b2r, g, b)


def _fc_body(h_ref, w_ref, o_ref):
    o_ref[...] = lax.dot_general(
        w_ref[...], h_ref[...], (((1,), (1,)), ((), ())),
        preferred_element_type=jnp.float32)


def _fc_matmul(h2, fc_w, fc_b):
    M = 8
    hp = jnp.zeros((M, N_NODES), jnp.float32).at[:2].set(h2)
    OB = 400
    out_t = pl.pallas_call(
        _fc_body,
        grid=(N_NODES // OB,),
        in_specs=[
            pl.BlockSpec((M, N_NODES), lambda i: (0, 0)),
            pl.BlockSpec((OB, N_NODES), lambda i: (i, 0)),
        ],
        out_specs=pl.BlockSpec((OB, M), lambda i: (i, 0)),
        out_shape=jax.ShapeDtypeStruct((N_NODES, M), jnp.float32),
    )(hp, fc_w)
    return out_t.T[:2] + fc_b[None, :]


def _tconv_mat(w, b, Ti, B):
    To = Ti - KSZ + 1
    H = w.shape[0]
    wm = w[:, 0, 0, :]
    d = jnp.arange(Ti)[:, None] - jnp.arange(To)[None, :]
    mask = (d >= 0) & (d < KSZ)
    take = jnp.take(wm, jnp.clip(d, 0, KSZ - 1), axis=1)
    wsel = jnp.where(mask[None], take, 0.0)
    wsel = jnp.transpose(wsel, (1, 2, 0))
    eye = jnp.eye(B, dtype=jnp.float32)
    G = eye[:, None, :, None, None] * wsel[None, :, None, :, :]
    G = G.reshape(B * Ti, B * To * H)
    bias = jnp.tile(b, B * To)[None, :]
    return G, bias


def _tconv2_mat(w, b, Ti, B):
    To = Ti - KSZ + 1
    Co, H = w.shape[0], w.shape[1]
    wm = w[:, :, 0, :]
    d = jnp.arange(Ti)[:, None] - jnp.arange(To)[None, :]
    mask = (d >= 0) & (d < KSZ)
    take = jnp.take(wm, jnp.clip(d, 0, KSZ - 1), axis=2)
    wsel = jnp.where(mask[None, None], take, 0.0)
    wsel = jnp.transpose(wsel, (2, 1, 3, 0))
    eye = jnp.eye(B, dtype=jnp.float32)
    G = (eye[:, None, None, :, None, None] *
         wsel[None, :, :, None, :, :])
    G = G.reshape(B * Ti * H, B * To * Co)
    bias = jnp.tile(b, B * To)[None, :]
    return G, bias


def _mv(zs, rowg_flat, col_pad, nch):
    N, F = zs.shape
    Fc = F // nch
    z_flat = zs.reshape(N, nch, Fc).transpose(1, 0, 2).reshape(nch * N, Fc)
    zeros = jnp.zeros((NPAD, Fc), jnp.float32)
    y_flat = _sc_mv_call(z_flat, rowg_flat, col_pad, zeros, Fc, nch)
    y = y_flat.reshape(nch, NPAD, Fc)[:, :N].transpose(1, 0, 2).reshape(N, F)
    return y


def _stage(Xc, t1w, t1b, chW, chb, t2w, t2b, bng, bnb,
           rowg, col_pad, dis, negd2, Ti, nch):
    B = 2
    H = t1w.shape[1]
    To = Ti - KSZ + 1
    F = B * To * H
    FP = nch * 128
    padc = ((0, 0), (0, FP - F))

    def pc(a):
        return jnp.pad(a, padc)

    g1p, b1p = _tconv_mat(t1w[0], t1b[0], Ti, B)
    g1q, b1q = _tconv_mat(t1w[1], t1b[1], Ti, B)
    g1r, b1r = _tconv_mat(t1w[2], t1b[2], Ti, B)
    g1p, g1q, g1r = pc(g1p), pc(g1q), pc(g1r)
    b1p, b1q, b1r = pc(b1p), pc(b1q), pc(b1r)
    BT = B * To
    wa = jnp.kron(jnp.eye(BT, dtype=jnp.float32), chW[0])
    w1t = jnp.kron(jnp.eye(BT, dtype=jnp.float32), chW[1])
    w2t = jnp.kron(jnp.eye(BT, dtype=jnp.float32), chW[2])
    padb = ((0, FP - F), (0, FP - F))
    wa = jnp.pad(wa, padb)
    w1t = jnp.pad(w1t, padb)
    w2t = jnp.pad(w2t, padb)
    bch = pc(jnp.tile(chb, BT)[None, :])
    g2p, b2p = _tconv2_mat(t2w[0], t2b[0], To, B)
    g2q, b2q = _tconv2_mat(t2w[1], t2b[1], To, B)
    g2r, b2r = _tconv2_mat(t2w[2], t2b[2], To, B)
    padr = ((0, FP - F), (0, 0))
    g2p = jnp.pad(g2p, padr)
    g2q = jnp.pad(g2q, padr)
    g2r = jnp.pad(g2r, padr)

    rowg_flat = (rowg[None, :] +
                 (jnp.arange(nch, dtype=jnp.int32) * N_NODES)[:, None]
                 ).reshape(-1)

    z0, zs0 = _tconv_call(Xc, g1p, g1q, g1r, b1p, b1q, b1r, dis)
    y1 = _mv(zs0, rowg_flat, col_pad, nch)
    zs1 = _scale_call(y1, negd2)
    y2 = _mv(zs1, rowg_flat, col_pad, nch)
    return _comb_call(z0, y1, y2, dis, wa, w1t, w2t, bch,
                      g2p, g2q, g2r, b2p, b2q, b2r,
                      bng[:, None], bnb[:, None])


def kernel(x, edge_index, s1_t1_w, s1_t1_b, s1_ch_W, s1_ch_b, s1_t2_w,
           s1_t2_b, s1_bn_g, s1_bn_b, s2_t1_w, s2_t1_b, s2_ch_W, s2_ch_b,
           s2_t2_w, s2_t2_b, s2_bn_g, s2_bn_b, fconv_w, fconv_b, fc_w, fc_b):
    E = edge_index.shape[1]
    EP = _epad(E)
    row = edge_index[0]
    col = edge_index[1]
    selfm = row == col
    padi = jnp.full((EP - E,), TRASH, jnp.int32)
    col_pad = jnp.concatenate([jnp.where(selfm, TRASH, col), padi])
    row_deg = jnp.concatenate([jnp.where(selfm, TRASH, row), padi])
    rowg = jnp.concatenate([row, jnp.zeros((EP - E,), jnp.int32)])

    ones8 = jnp.ones((KE, 128), jnp.float32)
    zeros8 = jnp.zeros((NPAD, 128), jnp.float32)
    degp = _sc_deg_call(row_deg, ones8, zeros8)
    deg = degp[:NPAD, 0] + degp[NPAD:, 0]
    deg = deg[:N_NODES]
    dis = jnp.where(deg > 0, 1.0 / jnp.sqrt(jnp.maximum(deg, 1.0)), 0.0)
    disc = dis[:, None]
    negd2 = -(dis * dis)[:, None]

    Xc = jnp.transpose(x[:, :, :, 0], (2, 0, 1)).reshape(N_NODES, 18)
    h1 = _stage(Xc, s1_t1_w, s1_t1_b, s1_ch_W, s1_ch_b, s1_t2_w, s1_t2_b,
                s1_bn_g, s1_bn_b, rowg, col_pad, disc, negd2, 9, 4)
    h2 = _stage(h1, s2_t1_w, s2_t1_b, s2_ch_W, s2_ch_b, s2_t2_w, s2_t2_b,
                s2_bn_g, s2_bn_b, rowg, col_pad, disc, negd2, 5, 2)
    hb = (h2 * fconv_w[0, 0, 0, 0] + fconv_b[0]).T
    return _fc_matmul(hb, fc_w, fc_b)


# speedup vs baseline: 35.6017x; 1.2470x over previous
import functools

import jax
import jax.numpy as jnp
from jax import lax
from jax.experimental import pallas as pl
from jax.experimental.pallas import tpu as pltpu
from jax.experimental.pallas import tpu_sc as plsc

N_NODES = 10000
NPAD = 10112
TRASH = 10000
KSZ = 3
KE = 128
NSUB = 16

@functools.cache
def _mesh():
    return plsc.VectorSubcoreMesh(core_axis_name="c", subcore_axis_name="s",
                                  num_cores=2, num_subcores=NSUB)


def _epad(E):
    g = 32 * KE
    return -(-E // g) * g


def _sc_mv_call(z_flat, row_flat, col_pad, zeros_pad, Fc, nchunks):
    per_core = nchunks // 2
    EP = row_flat.shape[0] // nchunks
    esub = EP // NSUB
    iters = esub // KE
    rps = NPAD // NSUB

    def body(z_ref, row_ref, col_ref, zero_ref, y_ref, acc, row_va, col_va,
             row_vb, col_vb, gbufa, gbufb, sema, semb):
        c = lax.axis_index("c")
        s = lax.axis_index("s")
        emax = esub - KE

        def drain_a():
            pltpu.make_async_copy(z_ref.at[row_va], gbufa, sema).wait()

        def drain_b():
            pltpu.make_async_copy(z_ref.at[row_vb], gbufb, semb).wait()

        for jj in range(per_core):
            chunk = c * per_core + jj
            pltpu.sync_copy(zero_ref.at[pl.ds(s * rps, rps)],
                            acc.at[pl.ds(s * rps, rps)])
            plsc.subcore_barrier()

            eb0 = s * esub
            pltpu.sync_copy(row_ref.at[pl.ds(chunk * EP + eb0, KE)], row_va)
            pltpu.sync_copy(col_ref.at[pl.ds(eb0, KE)], col_va)
            pltpu.async_copy(z_ref.at[row_va], gbufa, sema)

            def it(j, carry):
                eb_b = s * esub + (2 * j + 1) * KE
                pltpu.sync_copy(row_ref.at[pl.ds(chunk * EP + eb_b, KE)],
                                row_vb)
                pltpu.sync_copy(col_ref.at[pl.ds(eb_b, KE)], col_vb)
                pltpu.async_copy(z_ref.at[row_vb], gbufb, semb)
                drain_a()
                pltpu.sync_copy(gbufa, acc.at[col_va], add=True)
                eb_a = s * esub + jnp.minimum((2 * j + 2) * KE, emax)
                pltpu.sync_copy(row_ref.at[pl.ds(chunk * EP + eb_a, KE)],
                                row_va)
                pltpu.sync_copy(col_ref.at[pl.ds(eb_a, KE)], col_va)
                pltpu.async_copy(z_ref.at[row_va], gbufa, sema)
                drain_b()
                pltpu.sync_copy(gbufb, acc.at[col_vb], add=True)
                return carry

            lax.fori_loop(0, iters // 2, it, 0)
            drain_a()
            plsc.subcore_barrier()
            pltpu.sync_copy(acc.at[pl.ds(s * rps, rps)],
                            y_ref.at[pl.ds(chunk * NPAD + s * rps, rps)])
            plsc.subcore_barrier()

    k = pl.kernel(
        body,
        out_type=jax.ShapeDtypeStruct((nchunks * NPAD, Fc), jnp.float32),
        mesh=_mesh(),
        scratch_types=[
            pltpu.VMEM_SHARED((NPAD, Fc), jnp.float32),
            pltpu.VMEM((KE,), jnp.int32),
            pltpu.VMEM((KE,), jnp.int32),
            pltpu.VMEM((KE,), jnp.int32),
            pltpu.VMEM((KE,), jnp.int32),
            pltpu.VMEM((KE, Fc), jnp.float32),
            pltpu.VMEM((KE, Fc), jnp.float32),
            pltpu.SemaphoreType.DMA,
            pltpu.SemaphoreType.DMA,
        ],
    )
    return k(z_flat, row_flat, col_pad, zeros_pad)


def _sc_deg_call(row_deg, ones_in, zeros_deg):
    EP = row_deg.shape[0]
    esub = EP // (2 * NSUB)
    iters = esub // KE
    rps = NPAD // NSUB

    def body(rows_ref, ones_ref, zero_ref, y_ref, acc, row_v, vbuf, sem):
        c = lax.axis_index("c")
        s = lax.axis_index("s")
        wid = c * NSUB + s
        pltpu.sync_copy(zero_ref.at[pl.ds(s * rps, rps)],
                        acc.at[pl.ds(s * rps, rps)])
        pltpu.sync_copy(ones_ref, vbuf)
        plsc.subcore_barrier()

        def it(i, carry):
            eb = wid * esub + i * KE
            pltpu.sync_copy(rows_ref.at[pl.ds(eb, KE)], row_v)
            pltpu.sync_copy(vbuf, acc.at[row_v], add=True)
            return carry

        lax.fori_loop(0, iters, it, 0)
        plsc.subcore_barrier()
        pltpu.sync_copy(acc.at[pl.ds(s * rps, rps)],
                        y_ref.at[pl.ds(c * NPAD + s * rps, rps)])

    k = pl.kernel(
        body,
        out_type=jax.ShapeDtypeStruct((2 * NPAD, 128), jnp.float32),
        mesh=_mesh(),
        scratch_types=[
            pltpu.VMEM_SHARED((NPAD, 128), jnp.float32),
            pltpu.VMEM((KE,), jnp.int32),
            pltpu.VMEM((KE, 128), jnp.float32),
            pltpu.SemaphoreType.DMA,
        ],
    )
    return k(row_deg, ones_in, zeros_deg)


_NBLK = 2000


def _rowspec(cols):
    return pl.BlockSpec((_NBLK, cols), lambda i: (i, 0))


def _fullspec(r, c):
    return pl.BlockSpec((r, c), lambda i: (0, 0))


def _tconv_body(x_ref, gp, gq, gr, bp, bq, br, dis_ref, z_ref, zs_ref):
    X = x_ref[...]
    P = jnp.dot(X, gp[...], preferred_element_type=jnp.float32) + bp[...]
    Q = jnp.dot(X, gq[...], preferred_element_type=jnp.float32) + bq[...]
    R = jnp.dot(X, gr[...], preferred_element_type=jnp.float32) + br[...]
    Z = jnp.maximum(P * jax.nn.sigmoid(Q) + R, 0.0)
    z_ref[...] = Z
    zs_ref[...] = Z * dis_ref[...]


def _tconv_call(X, gp, gq, gr, bp, bq, br, dis):
    Ci = X.shape[1]
    F = gp.shape[1]
    return pl.pallas_call(
        _tconv_body,
        grid=(N_NODES // _NBLK,),
        in_specs=[_rowspec(Ci), _fullspec(Ci, F), _fullspec(Ci, F),
                  _fullspec(Ci, F), _fullspec(1, F), _fullspec(1, F),
                  _fullspec(1, F), _rowspec(1)],
        out_specs=[_rowspec(F), _rowspec(F)],
        out_shape=[jax.ShapeDtypeStruct((N_NODES, F), jnp.float32)] * 2,
    )(X, gp, gq, gr, bp, bq, br, dis)


def _scale_body(y_ref, s_ref, o_ref):
    o_ref[...] = y_ref[...] * s_ref[...]


def _scale_call(y, s):
    F = y.shape[1]
    return pl.pallas_call(
        _scale_body,
        grid=(N_NODES // _NBLK,),
        in_specs=[_rowspec(F), _rowspec(1)],
        out_specs=_rowspec(F),
        out_shape=jax.ShapeDtypeStruct((N_NODES, F), jnp.float32),
    )(y, s)


def _comb_body(z0_ref, y1_ref, y2_ref, dis_ref, wa, w1, w2, bch,
               g2p, g2q, g2r, b2p, b2q, b2r, g_ref, b_ref, o_ref):
    z0 = z0_ref[...]
    dis = dis_ref[...]
    tx1 = -dis * y1_ref[...]
    tx2 = (-2.0 * dis) * y2_ref[...] - z0
    out = jnp.dot(z0, wa[...], preferred_element_type=jnp.float32)
    out = out + jnp.dot(tx1, w1[...], preferred_element_type=jnp.float32)
    out = out + jnp.dot(tx2, w2[...], preferred_element_type=jnp.float32)
    C = jnp.maximum(out + bch[...], 0.0)
    P = jnp.dot(C, g2p[...], preferred_element_type=jnp.float32) + b2p[...]
    Q = jnp.dot(C, g2q[...], preferred_element_type=jnp.float32) + b2q[...]
    R = jnp.dot(C, g2r[...], preferred_element_type=jnp.float32) + b2r[...]
    D = jnp.maximum(P * jax.nn.sigmoid(Q) + R, 0.0)
    m = jnp.mean(D, axis=1, keepdims=True)
    v = jnp.mean((D - m) ** 2, axis=1, keepdims=True)
    Yn = (D - m) / jnp.sqrt(v + 1e-5) * g_ref[...] + b_ref[...]
    o_ref[...] = jnp.maximum(Yn, 0.0)


def _comb_call(z0, y1, y2, dis, wa, w1, w2, bch, g2p, g2q, g2r,
               b2p, b2q, b2r, g, b):
    F = z0.shape[1]
    Do = g2p.shape[1]
    return pl.pallas_call(
        _comb_body,
        grid=(N_NODES // _NBLK,),
        in_specs=[_rowspec(F), _rowspec(F), _rowspec(F), _rowspec(1),
                  _fullspec(F, F), _fullspec(F, F), _fullspec(F, F),
                  _fullspec(1, F), _fullspec(F, Do), _fullspec(F, Do),
                  _fullspec(F, Do), _fullspec(1, Do), _fullspec(1, Do),
                  _fullspec(1, Do), _rowspec(1), _rowspec(1)],
        out_specs=_rowspec(Do),
        out_shape=jax.ShapeDtypeStruct((N_NODES, Do), jnp.float32),
    )(z0, y1, y2, dis, wa, w1, w2, bch, g2p, g2q, g2r, b2p, b2q, b2r, g, b)


def _fc_body(h_ref, w_ref, o_ref):
    o_ref[...] = lax.dot_general(
        w_ref[...], h_ref[...], (((1,), (1,)), ((), ())),
        preferred_element_type=jnp.float32)


def _fc_matmul(h2, fc_w, fc_b):
    M = 8
    hp = jnp.zeros((M, N_NODES), jnp.float32).at[:2].set(h2)
    OB = 400
    out_t = pl.pallas_call(
        _fc_body,
        grid=(N_NODES // OB,),
        in_specs=[
            pl.BlockSpec((M, N_NODES), lambda i: (0, 0)),
            pl.BlockSpec((OB, N_NODES), lambda i: (i, 0)),
        ],
        out_specs=pl.BlockSpec((OB, M), lambda i: (i, 0)),
        out_shape=jax.ShapeDtypeStruct((N_NODES, M), jnp.float32),
    )(hp, fc_w)
    return out_t.T[:2] + fc_b[None, :]


def _tconv_mat(w, b, Ti, B):
    To = Ti - KSZ + 1
    H = w.shape[0]
    wm = w[:, 0, 0, :]
    d = jnp.arange(Ti)[:, None] - jnp.arange(To)[None, :]
    mask = (d >= 0) & (d < KSZ)
    take = jnp.take(wm, jnp.clip(d, 0, KSZ - 1), axis=1)
    wsel = jnp.where(mask[None], take, 0.0)
    wsel = jnp.transpose(wsel, (1, 2, 0))
    eye = jnp.eye(B, dtype=jnp.float32)
    G = eye[:, None, :, None, None] * wsel[None, :, None, :, :]
    G = G.reshape(B * Ti, B * To * H)
    bias = jnp.tile(b, B * To)[None, :]
    return G, bias


def _tconv2_mat(w, b, Ti, B):
    To = Ti - KSZ + 1
    Co, H = w.shape[0], w.shape[1]
    wm = w[:, :, 0, :]
    d = jnp.arange(Ti)[:, None] - jnp.arange(To)[None, :]
    mask = (d >= 0) & (d < KSZ)
    take = jnp.take(wm, jnp.clip(d, 0, KSZ - 1), axis=2)
    wsel = jnp.where(mask[None, None], take, 0.0)
    wsel = jnp.transpose(wsel, (2, 1, 3, 0))
    eye = jnp.eye(B, dtype=jnp.float32)
    G = (eye[:, None, None, :, None, None] *
         wsel[None, :, :, None, :, :])
    G = G.reshape(B * Ti * H, B * To * Co)
    bias = jnp.tile(b, B * To)[None, :]
    return G, bias


def _mv(zs, rowg_flat, col_pad, nch):
    N, F = zs.shape
    Fc = F // nch
    z_flat = zs.reshape(N, nch, Fc).transpose(1, 0, 2).reshape(nch * N, Fc)
    zeros = jnp.zeros((NPAD, Fc), jnp.float32)
    y_flat = _sc_mv_call(z_flat, rowg_flat, col_pad, zeros, Fc, nch)
    y = y_flat.reshape(nch, NPAD, Fc)[:, :N].transpose(1, 0, 2).reshape(N, F)
    return y


def _stage(Xc, t1w, t1b, chW, chb, t2w, t2b, bng, bnb,
           rowg, col_pad, dis, negd2, Ti, nch):
    B = 2
    H = t1w.shape[1]
    To = Ti - KSZ + 1
    F = B * To * H
    FP = nch * 128
    padc = ((0, 0), (0, FP - F))

    def pc(a):
        return jnp.pad(a, padc)

    g1p, b1p = _tconv_mat(t1w[0], t1b[0], Ti, B)
    g1q, b1q = _tconv_mat(t1w[1], t1b[1], Ti, B)
    g1r, b1r = _tconv_mat(t1w[2], t1b[2], Ti, B)
    g1p, g1q, g1r = pc(g1p), pc(g1q), pc(g1r)
    b1p, b1q, b1r = pc(b1p), pc(b1q), pc(b1r)
    BT = B * To
    wa = jnp.kron(jnp.eye(BT, dtype=jnp.float32), chW[0])
    w1t = jnp.kron(jnp.eye(BT, dtype=jnp.float32), chW[1])
    w2t = jnp.kron(jnp.eye(BT, dtype=jnp.float32), chW[2])
    padb = ((0, FP - F), (0, FP - F))
    wa = jnp.pad(wa, padb)
    w1t = jnp.pad(w1t, padb)
    w2t = jnp.pad(w2t, padb)
    bch = pc(jnp.tile(chb, BT)[None, :])
    g2p, b2p = _tconv2_mat(t2w[0], t2b[0], To, B)
    g2q, b2q = _tconv2_mat(t2w[1], t2b[1], To, B)
    g2r, b2r = _tconv2_mat(t2w[2], t2b[2], To, B)
    padr = ((0, FP - F), (0, 0))
    g2p = jnp.pad(g2p, padr)
    g2q = jnp.pad(g2q, padr)
    g2r = jnp.pad(g2r, padr)

    rowg_flat = (rowg[None, :] +
                 (jnp.arange(nch, dtype=jnp.int32) * N_NODES)[:, None]
                 ).reshape(-1)

    z0, zs0 = _tconv_call(Xc, g1p, g1q, g1r, b1p, b1q, b1r, dis)
    y1 = _mv(zs0, rowg_flat, col_pad, nch)
    zs1 = _scale_call(y1, negd2)
    y2 = _mv(zs1, rowg_flat, col_pad, nch)
    return _comb_call(z0, y1, y2, dis, wa, w1t, w2t, bch,
                      g2p, g2q, g2r, b2p, b2q, b2r,
                      bng[:, None], bnb[:, None])


def kernel(x, edge_index, s1_t1_w, s1_t1_b, s1_ch_W, s1_ch_b, s1_t2_w,
           s1_t2_b, s1_bn_g, s1_bn_b, s2_t1_w, s2_t1_b, s2_ch_W, s2_ch_b,
           s2_t2_w, s2_t2_b, s2_bn_g, s2_bn_b, fconv_w, fconv_b, fc_w, fc_b):
    E = edge_index.shape[1]
    EP = _epad(E)
    row = edge_index[0]
    col = edge_index[1]
    selfm = row == col
    padi = jnp.full((EP - E,), TRASH, jnp.int32)
    col_pad = jnp.concatenate([jnp.where(selfm, TRASH, col), padi])
    row_deg = jnp.concatenate([jnp.where(selfm, TRASH, row), padi])
    rowg = jnp.concatenate([row, jnp.zeros((EP - E,), jnp.int32)])

    ones8 = jnp.ones((KE, 128), jnp.float32)
    zeros8 = jnp.zeros((NPAD, 128), jnp.float32)
    degp = _sc_deg_call(row_deg, ones8, zeros8)
    deg = degp[:NPAD, 0] + degp[NPAD:, 0]
    deg = deg[:N_NODES]
    dis = jnp.where(deg > 0, 1.0 / jnp.sqrt(jnp.maximum(deg, 1.0)), 0.0)
    disc = dis[:, None]
    negd2 = -(dis * dis)[:, None]

    Xc = jnp.transpose(x[:, :, :, 0], (2, 0, 1)).reshape(N_NODES, 18)
    h1 = _stage(Xc, s1_t1_w, s1_t1_b, s1_ch_W, s1_ch_b, s1_t2_w, s1_t2_b,
                s1_bn_g, s1_bn_b, rowg, col_pad, disc, negd2, 9, 4)
    h2 = _stage(h1, s2_t1_w, s2_t1_b, s2_ch_W, s2_ch_b, s2_t2_w, s2_t2_b,
                s2_bn_g, s2_bn_b, rowg, col_pad, disc, negd2, 5, 2)
    hb = (h2 * fconv_w[0, 0, 0, 0] + fconv_b[0]).T
    return _fc_matmul(hb, fc_w, fc_b)
